# baseline (speedup 1.0000x reference)
import jax
import jax.numpy as jnp
from jax import lax
from jax.experimental import pallas as pl
from jax.experimental.pallas import tpu as pltpu
from jax.experimental.pallas import tpu_sc as plsc

D = 32
K = 8
ND = D * K
W = 2 * ND + K
N = 16384

NC = 2
NS = 16
NW = NC * NS
CW = 4096
NCH = N // CW

BN = 2048


def _sc_mean(xt_hbm, mean_hbm, b0, b1, si0, si1, so0, so1):
    wid = lax.axis_index("s") * NC + lax.axis_index("c")
    r0 = wid * 8
    BUF, SI, SO = [b0, b1], [si0, si1], [so0, so1]
    hin, hout = {}, {}

    def start_in(c):
        b = c & 1
        h = pltpu.make_async_copy(
            xt_hbm.at[pl.ds(r0, 8), pl.ds(c * CW, CW)], BUF[b], SI[b])
        h.start()
        hin[c] = h

    start_in(0)
    start_in(1)
    for c in range(NCH):
        b = c & 1
        hin[c].wait()
        if c >= 2:
            hout[c - 2].wait()
        h = pltpu.make_async_copy(
            BUF[b], mean_hbm.at[pl.ds(r0, 8), pl.ds(c * CW, CW)], SO[b])
        h.start()
        hout[c] = h
        if c + 2 < NCH:
            start_in(c + 2)
    hout[NCH - 2].wait()
    hout[NCH - 1].wait()


def _tc_body(xs_ref, xp_ref, std_ref, pi_ref):
    std_ref[...] = jax.nn.sigmoid(xs_ref[...])
    logits = xp_ref[...]
    m = jnp.max(logits, axis=0, keepdims=True)
    e = jnp.exp(logits - m)
    s = jnp.sum(e, axis=0, keepdims=True)
    pi_ref[...] = e / s


def kernel(x):
    xt = jnp.transpose(x)

    mean_t = pl.kernel(
        _sc_mean,
        mesh=plsc.VectorSubcoreMesh(core_axis_name="c", subcore_axis_name="s"),
        out_type=jax.ShapeDtypeStruct((ND, N), jnp.float32),
        scratch_types=[
            pltpu.VMEM((8, CW), jnp.float32),
            pltpu.VMEM((8, CW), jnp.float32),
            pltpu.SemaphoreType.DMA,
            pltpu.SemaphoreType.DMA,
            pltpu.SemaphoreType.DMA,
            pltpu.SemaphoreType.DMA,
        ],
        compiler_params=pltpu.CompilerParams(needs_layout_passes=False),
    )(xt)

    std_t, pi_t = pl.pallas_call(
        _tc_body,
        grid=(N // BN,),
        in_specs=[
            pl.BlockSpec((ND, BN), lambda j: (1, j)),
            pl.BlockSpec((K, BN), lambda j: (2 * ND // K, j)),
        ],
        out_specs=[
            pl.BlockSpec((ND, BN), lambda j: (0, j)),
            pl.BlockSpec((K, BN), lambda j: (0, j)),
        ],
        out_shape=[
            jax.ShapeDtypeStruct((ND, N), jnp.float32),
            jax.ShapeDtypeStruct((K, N), jnp.float32),
        ],
    )(xt, xt)

    mean = jnp.transpose(mean_t).reshape(N, D, K)
    std = jnp.transpose(std_t).reshape(N, D, K)
    pi = jnp.transpose(pi_t)
    return (mean, std, pi)


# speedup vs baseline: 1.7223x; 1.7223x over previous
import jax
import jax.numpy as jnp
from jax import lax
from jax.experimental import pallas as pl

D = 32
K = 8
ND = D * K
W = 2 * ND + K
N = 16384

BN = 4096


def _tc_body(xm_ref, xs_ref, xp_ref, mean_ref, std_ref, pi_ref):
    mean_ref[...] = xm_ref[...]
    std_ref[...] = jax.nn.sigmoid(xs_ref[...])
    logits = xp_ref[...]
    m = jnp.max(logits, axis=0, keepdims=True)
    e = jnp.exp(logits - m)
    s = jnp.sum(e, axis=0, keepdims=True)
    pi_ref[...] = e / s


def kernel(x):
    xt = jnp.transpose(x)
    mean_t, std_t, pi_t = pl.pallas_call(
        _tc_body,
        grid=(N // BN,),
        in_specs=[
            pl.BlockSpec((ND, BN), lambda j: (0, j)),
            pl.BlockSpec((ND, BN), lambda j: (1, j)),
            pl.BlockSpec((K, BN), lambda j: (2 * ND // K, j)),
        ],
        out_specs=[
            pl.BlockSpec((ND, BN), lambda j: (0, j)),
            pl.BlockSpec((ND, BN), lambda j: (0, j)),
            pl.BlockSpec((K, BN), lambda j: (0, j)),
        ],
        out_shape=[
            jax.ShapeDtypeStruct((ND, N), jnp.float32),
            jax.ShapeDtypeStruct((ND, N), jnp.float32),
            jax.ShapeDtypeStruct((K, N), jnp.float32),
        ],
    )(xt, xt, xt)
    mean = jnp.transpose(mean_t).reshape(N, D, K)
    std = jnp.transpose(std_t).reshape(N, D, K)
    pi = jnp.transpose(pi_t)
    return (mean, std, pi)
